# baseline (speedup 1.0000x reference)
import jax
import jax.numpy as jnp
from jax.experimental import pallas as pl


def _fused_mlp_kernel(x_ref, w1_ref, b1_ref, w2_ref, b2_ref, s_ref, o_ref):
    x = x_ref[...]
    h = jnp.dot(x.astype(jnp.bfloat16), w1_ref[...],
                preferred_element_type=jnp.float32)
    h = jnp.maximum(h + b1_ref[...], 0.0).astype(jnp.bfloat16)
    y = jnp.dot(h, w2_ref[...], preferred_element_type=jnp.float32)
    y = y + b2_ref[...] + x
    var = jnp.mean(y * y, axis=-1, keepdims=True)
    o_ref[...] = (y * jax.lax.rsqrt(var + 1e-6)) * s_ref[...]


def kernel(x, W1, b1, W2, b2, ln_scale):
    B, S, D = x.shape
    K, V = W1.shape
    M = B * S
    TM = 512

    xf = x.reshape(M, D)
    w1b = W1.astype(jnp.bfloat16)
    w2b = W2.astype(jnp.bfloat16)
    b1r = b1.reshape(1, V)
    b2r = b2.reshape(1, D)
    snr = ln_scale.reshape(1, D)

    out = pl.pallas_call(
        _fused_mlp_kernel,
        grid=(M // TM,),
        in_specs=[
            pl.BlockSpec((TM, D), lambda m: (m, 0)),
            pl.BlockSpec((K, V), lambda m: (0, 0)),
            pl.BlockSpec((1, V), lambda m: (0, 0)),
            pl.BlockSpec((V, D), lambda m: (0, 0)),
            pl.BlockSpec((1, D), lambda m: (0, 0)),
            pl.BlockSpec((1, D), lambda m: (0, 0)),
        ],
        out_specs=pl.BlockSpec((TM, D), lambda m: (m, 0)),
        out_shape=jax.ShapeDtypeStruct((M, D), jnp.float32),
    )(xf, w1b, b1r, w2b, b2r, snr)
    return out.reshape(B, S, D)


# speedup vs baseline: 1.1482x; 1.1482x over previous
import jax
import jax.numpy as jnp
from jax.experimental import pallas as pl


def _fused_mlp_kernel(x_ref, w1_ref, b1_ref, w2_ref, b2_ref, s_ref, o_ref):
    x = x_ref[...]
    h = jnp.dot(x.astype(jnp.bfloat16), w1_ref[...].astype(jnp.bfloat16),
                preferred_element_type=jnp.float32)
    h = jnp.maximum(h + b1_ref[...], 0.0).astype(jnp.bfloat16)
    y = jnp.dot(h, w2_ref[...].astype(jnp.bfloat16),
                preferred_element_type=jnp.float32)
    y = y + b2_ref[...] + x
    var = jnp.mean(y * y, axis=-1, keepdims=True)
    o_ref[...] = (y * jax.lax.rsqrt(var + 1e-6)) * s_ref[...]


def kernel(x, W1, b1, W2, b2, ln_scale):
    B, S, D = x.shape
    K, V = W1.shape
    M = B * S
    TM = 1024

    xf = x.reshape(M, D)
    b1r = b1.reshape(1, V)
    b2r = b2.reshape(1, D)
    snr = ln_scale.reshape(1, D)

    out = pl.pallas_call(
        _fused_mlp_kernel,
        grid=(M // TM,),
        in_specs=[
            pl.BlockSpec((TM, D), lambda m: (m, 0)),
            pl.BlockSpec((K, V), lambda m: (0, 0)),
            pl.BlockSpec((1, V), lambda m: (0, 0)),
            pl.BlockSpec((V, D), lambda m: (0, 0)),
            pl.BlockSpec((1, D), lambda m: (0, 0)),
            pl.BlockSpec((1, D), lambda m: (0, 0)),
        ],
        out_specs=pl.BlockSpec((TM, D), lambda m: (m, 0)),
        out_shape=jax.ShapeDtypeStruct((M, D), jnp.float32),
    )(xf, W1, b1r, W2, b2r, snr)
    return out.reshape(B, S, D)
